# baseline (speedup 1.0000x reference)
import functools

import jax
import jax.numpy as jnp
from jax import lax
from jax.experimental import pallas as pl
from jax.experimental.pallas import tpu as pltpu
from jax.experimental.pallas import tpu_sc as plsc

B = 16384
D = 64
BQ = B // 4
DQ = 4 * D

_info = plsc.get_sparse_core_info()
_NC, _NS, _L = _info.num_cores, _info.num_subcores, _info.num_lanes
_NW = _NC * _NS
_QPW = BQ // _NW
_IPW = B // _NW
_HQ = _QPW // 2

_mesh = plsc.VectorSubcoreMesh(core_axis_name="c", subcore_axis_name="s")


@functools.partial(
    pl.kernel,
    mesh=_mesh,
    out_type=jax.ShapeDtypeStruct((BQ, DQ), jnp.float32),
    compiler_params=pltpu.CompilerParams(
        needs_layout_passes=False,
        disable_bounds_checks=True,
        disable_semaphore_checks=True,
    ),
    scratch_types=[
        pltpu.VMEM((_IPW,), jnp.int32),
        pltpu.VMEM((_QPW,), jnp.int32),
        pltpu.VMEM((_HQ, DQ), jnp.float32),
        pltpu.VMEM((_HQ, DQ), jnp.float32),
        pltpu.SemaphoreType.DMA,
        pltpu.SemaphoreType.DMA,
        pltpu.SemaphoreType.DMA,
    ],
)
def _gather_kernel(quads_hbm, idx_hbm, out_hbm, idx_v, qidx_v,
                   rows0_v, rows1_v, sg0, sg1, sw):
    wid = lax.axis_index("s") * _NC + lax.axis_index("c")
    pltpu.sync_copy(idx_hbm.at[pl.ds(wid * _IPW, _IPW)], idx_v)
    lane4 = lax.iota(jnp.int32, _L) * 4
    for g in range(_QPW // _L):
        i0 = plsc.load_gather(idx_v, [lane4 + (4 * _L) * g])
        i1 = plsc.load_gather(idx_v, [lane4 + ((4 * _L) * g + 1)])
        i2 = plsc.load_gather(idx_v, [lane4 + ((4 * _L) * g + 2)])
        i3 = plsc.load_gather(idx_v, [lane4 + ((4 * _L) * g + 3)])
        qidx_v[pl.ds(g * _L, _L)] = i0 * 64 + i1 * 16 + i2 * 4 + i3
    g0 = pltpu.async_copy(quads_hbm.at[qidx_v.at[pl.ds(0, _HQ)]], rows0_v, sg0)
    g1 = pltpu.async_copy(quads_hbm.at[qidx_v.at[pl.ds(_HQ, _HQ)]], rows1_v, sg1)
    base = wid * _QPW
    g0.wait()
    w0 = pltpu.async_copy(rows0_v, out_hbm.at[pl.ds(base, _HQ)], sw)
    g1.wait()
    w1 = pltpu.async_copy(rows1_v, out_hbm.at[pl.ds(base + _HQ, _HQ)], sw)
    w0.wait()
    w1.wait()


def kernel(a, emb):
    pairs = jnp.concatenate(
        [jnp.repeat(emb, 4, axis=0), jnp.tile(emb, (4, 1))], axis=-1
    )
    quads = jnp.concatenate(
        [jnp.repeat(pairs, 16, axis=0), jnp.tile(pairs, (16, 1))], axis=-1
    )
    out = _gather_kernel(quads, a.astype(jnp.int32))
    return out.reshape(B, D)[:, None, :]


# speedup vs baseline: 1.0165x; 1.0165x over previous
import functools

import jax
import jax.numpy as jnp
from jax import lax
from jax.experimental import pallas as pl
from jax.experimental.pallas import tpu as pltpu
from jax.experimental.pallas import tpu_sc as plsc

B = 16384
D = 64
BQ = B // 4
DQ = 4 * D

_info = plsc.get_sparse_core_info()
_NC, _NS, _L = _info.num_cores, _info.num_subcores, _info.num_lanes
_NW = _NC * _NS
_QPW = BQ // _NW
_IPW = B // _NW

_mesh = plsc.VectorSubcoreMesh(core_axis_name="c", subcore_axis_name="s")


@functools.partial(
    pl.kernel,
    mesh=_mesh,
    out_type=jax.ShapeDtypeStruct((BQ, DQ), jnp.float32),
    compiler_params=pltpu.CompilerParams(
        needs_layout_passes=False,
        use_tc_tiling_on_sc=False,
    ),
    scratch_types=[
        pltpu.VMEM((_IPW,), jnp.int32),
        pltpu.VMEM((_QPW,), jnp.int32),
        pltpu.VMEM((_QPW, DQ), jnp.float32),
        pltpu.SemaphoreType.DMA,
    ],
)
def _gather_kernel(quads_hbm, idx_hbm, out_hbm, idx_v, qidx_v, rows_v, sem):
    wid = lax.axis_index("s") * _NC + lax.axis_index("c")
    pltpu.sync_copy(idx_hbm.at[pl.ds(wid * _IPW, _IPW)], idx_v)
    lane4 = lax.iota(jnp.int32, _L) * 4
    for g in range(_QPW // _L):
        i0 = plsc.load_gather(idx_v, [lane4 + (4 * _L) * g])
        i1 = plsc.load_gather(idx_v, [lane4 + ((4 * _L) * g + 1)])
        i2 = plsc.load_gather(idx_v, [lane4 + ((4 * _L) * g + 2)])
        i3 = plsc.load_gather(idx_v, [lane4 + ((4 * _L) * g + 3)])
        qidx_v[pl.ds(g * _L, _L)] = i0 * 64 + i1 * 16 + i2 * 4 + i3
    pltpu.async_copy(quads_hbm.at[qidx_v], rows_v, sem).wait()
    pltpu.sync_copy(rows_v, out_hbm.at[pl.ds(wid * _QPW, _QPW)])


def kernel(a, emb):
    pairs = jnp.concatenate(
        [jnp.repeat(emb, 4, axis=0), jnp.tile(emb, (4, 1))], axis=-1
    )
    quads = jnp.concatenate(
        [jnp.repeat(pairs, 16, axis=0), jnp.tile(pairs, (16, 1))], axis=-1
    )
    out = _gather_kernel(quads, a.astype(jnp.int32))
    return out.reshape(B, D)[:, None, :]


# speedup vs baseline: 1.0258x; 1.0092x over previous
import functools

import jax
import jax.numpy as jnp
from jax import lax
from jax.experimental import pallas as pl
from jax.experimental.pallas import tpu as pltpu
from jax.experimental.pallas import tpu_sc as plsc

B = 16384
D = 64
BQ = B // 4
DQ = 4 * D

_info = plsc.get_sparse_core_info()
_NC, _NS, _L = _info.num_cores, _info.num_subcores, _info.num_lanes
_NW = _NC * _NS
_QPW = BQ // _NW
_IPW = B // _NW

_mesh = plsc.VectorSubcoreMesh(core_axis_name="c", subcore_axis_name="s")


@functools.partial(
    pl.kernel,
    mesh=_mesh,
    out_type=jax.ShapeDtypeStruct((BQ, DQ), jnp.float32),
    compiler_params=pltpu.CompilerParams(
        needs_layout_passes=False,
        use_tc_tiling_on_sc=False,
        skip_device_barrier=True,
    ),
    scratch_types=[
        pltpu.VMEM((_IPW,), jnp.int32),
        pltpu.VMEM((_QPW,), jnp.int32),
        pltpu.VMEM((_QPW, DQ), jnp.float32),
        pltpu.SemaphoreType.DMA,
    ],
)
def _gather_kernel(quads_hbm, idx_hbm, out_hbm, idx_v, qidx_v, rows_v, sem):
    wid = lax.axis_index("s") * _NC + lax.axis_index("c")
    pltpu.sync_copy(idx_hbm.at[pl.ds(wid * _IPW, _IPW)], idx_v)
    lane4 = lax.iota(jnp.int32, _L) * 4
    for g in range(_QPW // _L):
        i0 = plsc.load_gather(idx_v, [lane4 + (4 * _L) * g])
        i1 = plsc.load_gather(idx_v, [lane4 + ((4 * _L) * g + 1)])
        i2 = plsc.load_gather(idx_v, [lane4 + ((4 * _L) * g + 2)])
        i3 = plsc.load_gather(idx_v, [lane4 + ((4 * _L) * g + 3)])
        qidx_v[pl.ds(g * _L, _L)] = i0 * 64 + i1 * 16 + i2 * 4 + i3
    pltpu.async_copy(quads_hbm.at[qidx_v], rows_v, sem).wait()
    pltpu.sync_copy(rows_v, out_hbm.at[pl.ds(wid * _QPW, _QPW)])


def kernel(a, emb):
    pairs = jnp.concatenate(
        [jnp.repeat(emb, 4, axis=0), jnp.tile(emb, (4, 1))], axis=-1
    )
    quads = jnp.concatenate(
        [jnp.repeat(pairs, 16, axis=0), jnp.tile(pairs, (16, 1))], axis=-1
    )
    out = _gather_kernel(quads, a.astype(jnp.int32))
    return out.reshape(B, D)[:, None, :]
